# baseline (speedup 1.0000x reference)
import functools

import jax
import jax.numpy as jnp
from jax import lax
from jax.experimental import pallas as pl
from jax.experimental.pallas import tpu as pltpu
from jax.experimental.pallas import tpu_sc as plsc

NF = 26
FD = 38462
D = 16
B = 16384
NC = 2
NS = 16
NW = NC * NS
RW = B // NW
NQ = RW // 128
CH = 2404
L = 16


def kernel(x, table, bias):
    xT = x.T

    mesh = plsc.VectorSubcoreMesh(core_axis_name="c", subcore_axis_name="s")

    @functools.partial(
        pl.kernel,
        mesh=mesh,
        out_type=jax.ShapeDtypeStruct((B, D), jnp.float32),
        compiler_params=pltpu.CompilerParams(use_tc_tiling_on_sc=False),
        scratch_types=[
            pltpu.VMEM((NF * NQ, 128), jnp.int32),
            pltpu.VMEM((RW, D), jnp.float32),
            pltpu.VMEM((D,), jnp.float32),
            pltpu.VMEM_SHARED((FD, D), jnp.float32),
            pltpu.VMEM_SHARED((FD, D), jnp.float32),
            pltpu.SemaphoreType.DMA,
            pltpu.SemaphoreType.DMA,
            pltpu.SemaphoreType.DMA,
        ],
    )
    def k(xT_hbm, table_hbm, bias_hbm, out_hbm,
          idxT, acc, bias_v, slab0, slab1, gsem, isem, ssem):
        cid = lax.axis_index("c")
        sid = lax.axis_index("s")
        wid = sid * NC + cid
        base = wid * RW
        slabs = (slab0, slab1)

        stages = [
            pltpu.async_copy(
                xT_hbm.at[f, pl.ds(base + q * 128, 128)],
                idxT.at[f * NQ + q], isem)
            for f in range(NF) for q in range(NQ)
        ]

        coff = lax.min(sid * CH, FD - CH)

        def slab_copy(f, buf):
            return pltpu.async_copy(
                table_hbm.at[pl.ds(f * FD + coff, CH), :],
                buf.at[pl.ds(coff, CH), :], ssem)

        pltpu.sync_copy(bias_hbm, bias_v)
        bias_vec = bias_v[...]

        c0 = slab_copy(0, slabs[0])

        def seed(b, c2):
            acc[b, :] = bias_vec
            return c2
        lax.fori_loop(0, RW, seed, 0)

        for c in stages:
            c.wait()
        c0.wait()
        plsc.subcore_barrier()

        for f in range(NF):
            buf = slabs[f % 2]
            cnext = slab_copy(f + 1, slabs[(f + 1) % 2]) if f + 1 < NF else None
            gathers = [
                pltpu.async_copy(buf.at[idxT.at[f * NQ + q]],
                                 acc.at[pl.ds(q * 128, 128)], gsem, add=True)
                for q in range(NQ)
            ]
            for c in gathers:
                c.wait()
            if cnext is not None:
                cnext.wait()
            plsc.subcore_barrier()

        pltpu.sync_copy(acc, out_hbm.at[pl.ds(base, RW)])

    return k(xT, table, bias)


# speedup vs baseline: 1.0698x; 1.0698x over previous
import functools

import jax
import jax.numpy as jnp
import numpy as np
from jax import lax
from jax.experimental import pallas as pl
from jax.experimental.pallas import tpu as pltpu
from jax.experimental.pallas import tpu_sc as plsc

_FIELD_DIMS = [38462] * 26
_OFFSETS = np.concatenate([[0], np.cumsum(_FIELD_DIMS[:-1])]).astype(np.int32)

NF = 26
FD = 38462
D = 16
B = 16384
NC = 2
NS = 16
NW = NC * NS
RW = B // NW
NQ = RW // 128
CH = 2404
L = 16
SPF = 14
NHF = NF - SPF


def kernel(x, table, bias):
    xT = x.T

    mesh = plsc.VectorSubcoreMesh(core_axis_name="c", subcore_axis_name="s")

    @functools.partial(
        pl.kernel,
        mesh=mesh,
        out_type=jax.ShapeDtypeStruct((B, D), jnp.float32),
        compiler_params=pltpu.CompilerParams(use_tc_tiling_on_sc=False),
        scratch_types=[
            pltpu.VMEM((NF * NQ, 128), jnp.int32),
            pltpu.VMEM((RW, D), jnp.float32),
            pltpu.VMEM((D,), jnp.float32),
            pltpu.VMEM_SHARED((FD, D), jnp.float32),
            pltpu.VMEM_SHARED((FD, D), jnp.float32),
            pltpu.SemaphoreType.DMA,
            pltpu.SemaphoreType.DMA,
            pltpu.SemaphoreType.DMA,
            pltpu.SemaphoreType.DMA,
        ],
    )
    def k(xT_hbm, table_hbm, bias_hbm, out_hbm,
          idxT, acc, bias_v, slab0, slab1, gsem, g2sem, isem, ssem):
        cid = lax.axis_index("c")
        sid = lax.axis_index("s")
        wid = sid * NC + cid
        base = wid * RW
        slabs = (slab0, slab1)

        stages = [
            pltpu.async_copy(
                xT_hbm.at[f, pl.ds(base + q * 128, 128)],
                idxT.at[f * NQ + q], isem)
            for f in range(NF) for q in range(NQ)
        ]

        coff = lax.min(sid * CH, FD - CH)

        def slab_copy(f, buf):
            return pltpu.async_copy(
                table_hbm.at[pl.ds(f * FD + coff, CH), :],
                buf.at[pl.ds(coff, CH), :], ssem)

        pltpu.sync_copy(bias_hbm, bias_v)
        bias_vec = bias_v[...]

        def seed(b, c2):
            acc[b, :] = bias_vec
            return c2
        lax.fori_loop(0, RW, seed, 0)

        for c in stages:
            c.wait()

        for f in range(NHF):
            off = int(_OFFSETS[f])
            def add_off(t, c2, f=f, off=off):
                q = t // 8
                col = (t % 8) * L
                r = f * NQ + q
                idxT[r, pl.ds(col, L)] = idxT[r, pl.ds(col, L)] + off
                return c2
            lax.fori_loop(0, 8 * NQ, add_off, 0)

        hbm_gathers = [
            pltpu.async_copy(table_hbm.at[idxT.at[f * NQ + q]],
                             acc.at[pl.ds(q * 128, 128)], gsem, add=True)
            for f in range(NHF) for q in range(NQ)
        ]

        c0 = slab_copy(NHF, slabs[0])
        c0.wait()
        plsc.subcore_barrier()

        for i in range(SPF):
            f = NHF + i
            buf = slabs[i % 2]
            cnext = slab_copy(f + 1, slabs[(i + 1) % 2]) if i + 1 < SPF else None
            gathers = [
                pltpu.async_copy(buf.at[idxT.at[f * NQ + q]],
                                 acc.at[pl.ds(q * 128, 128)], g2sem, add=True)
                for q in range(NQ)
            ]
            for c in gathers:
                c.wait()
            if cnext is not None:
                cnext.wait()
            plsc.subcore_barrier()

        for c in hbm_gathers:
            c.wait()

        pltpu.sync_copy(acc, out_hbm.at[pl.ds(base, RW)])

    return k(xT, table, bias)


# speedup vs baseline: 1.1581x; 1.0826x over previous
import functools

import jax
import jax.numpy as jnp
import numpy as np
from jax import lax
from jax.experimental import pallas as pl
from jax.experimental.pallas import tpu as pltpu
from jax.experimental.pallas import tpu_sc as plsc

_FIELD_DIMS = [38462] * 26
_OFFSETS = np.concatenate([[0], np.cumsum(_FIELD_DIMS[:-1])]).astype(np.int32)

NF = 26
D = 16
B = 16384
NC = 2
NS = 16
NW = NC * NS
RW = B // NW
L = 16


def kernel(x, table, bias):
    xT = x.T

    mesh = plsc.VectorSubcoreMesh(core_axis_name="c", subcore_axis_name="s")

    @functools.partial(
        pl.kernel,
        mesh=mesh,
        out_type=jax.ShapeDtypeStruct((B, D), jnp.float32),
        compiler_params=pltpu.CompilerParams(use_tc_tiling_on_sc=False),
        scratch_types=[
            pltpu.VMEM((NF * RW,), jnp.int32),
            pltpu.VMEM((RW, D), jnp.float32),
            pltpu.VMEM((D,), jnp.float32),
            pltpu.SemaphoreType.DMA,
            pltpu.SemaphoreType.DMA,
        ],
    )
    def k(xT_hbm, table_hbm, bias_hbm, out_hbm, idxv, acc, bias_v, sem, sem2):
        wid = lax.axis_index("s") * NC + lax.axis_index("c")
        base = wid * RW
        pltpu.sync_copy(bias_hbm, bias_v)

        stages = [
            pltpu.async_copy(
                xT_hbm.at[f, pl.ds(base, RW)],
                idxv.at[pl.ds(f * RW, RW)], sem2)
            for f in range(NF)
        ]

        bias_vec = bias_v[...]
        def seed(b, c2):
            acc[b, :] = bias_vec
            return c2
        lax.fori_loop(0, RW, seed, 0)

        for c in stages:
            c.wait()

        for f in range(NF):
            off = int(_OFFSETS[f])
            def add_off(t, c2, f=f, off=off):
                sl = pl.ds(f * RW + t * L, L)
                idxv[sl] = idxv[sl] + off
                return c2
            lax.fori_loop(0, RW // L, add_off, 0)

        copies = [
            pltpu.async_copy(table_hbm.at[idxv.at[pl.ds(f * RW, RW)]],
                             acc.at[:], sem, add=True)
            for f in range(NF)
        ]
        for c in copies:
            c.wait()

        pltpu.sync_copy(acc, out_hbm.at[pl.ds(base, RW)])

    return k(xT, table, bias)


# speedup vs baseline: 1.1656x; 1.0064x over previous
import functools

import jax
import jax.numpy as jnp
import numpy as np
from jax import lax
from jax.experimental import pallas as pl
from jax.experimental.pallas import tpu as pltpu
from jax.experimental.pallas import tpu_sc as plsc

_FIELD_DIMS = [38462] * 26
_OFFSETS = np.concatenate([[0], np.cumsum(_FIELD_DIMS[:-1])]).astype(np.int32)

NF = 26
D = 16
B = 16384
NC = 2
NS = 16
NW = NC * NS
RW = B // NW
L = 16


def kernel(x, table, bias):
    xT = x.T

    mesh = plsc.VectorSubcoreMesh(core_axis_name="c", subcore_axis_name="s")

    @functools.partial(
        pl.kernel,
        mesh=mesh,
        out_type=jax.ShapeDtypeStruct((B, D), jnp.float32),
        compiler_params=pltpu.CompilerParams(use_tc_tiling_on_sc=False),
        scratch_types=[
            pltpu.VMEM((NF * RW,), jnp.int32),
            pltpu.VMEM((RW, D), jnp.float32),
            pltpu.VMEM((D,), jnp.float32),
            pltpu.SemaphoreType.DMA,
            pltpu.SemaphoreType.DMA,
        ],
    )
    def k(xT_hbm, table_hbm, bias_hbm, out_hbm, idxv, acc, bias_v, sem, sem2):
        wid = lax.axis_index("s") * NC + lax.axis_index("c")
        base = wid * RW
        pltpu.sync_copy(bias_hbm, bias_v)

        stages = [
            pltpu.async_copy(
                xT_hbm.at[f, pl.ds(base, RW)],
                idxv.at[pl.ds(f * RW, RW)], sem2)
            for f in range(NF)
        ]

        bias_vec = bias_v[...]
        def seed(b, c2):
            acc[b, :] = bias_vec
            return c2
        lax.fori_loop(0, RW, seed, 0)

        copies = []
        for f in range(NF):
            stages[f].wait()
            off = int(_OFFSETS[f])
            def add_off(t, c2, f=f, off=off):
                sl = pl.ds(f * RW + t * L, L)
                idxv[sl] = idxv[sl] + off
                return c2
            lax.fori_loop(0, RW // L, add_off, 0)
            copies.append(
                pltpu.async_copy(table_hbm.at[idxv.at[pl.ds(f * RW, RW)]],
                                 acc.at[:], sem, add=True))
        for c in copies:
            c.wait()

        pltpu.sync_copy(acc, out_hbm.at[pl.ds(base, RW)])

    return k(xT, table, bias)
